# Initial kernel scaffold; baseline (speedup 1.0000x reference)
import jax
import jax.numpy as jnp
from jax.experimental import pallas as pl


def kernel(f_maps, indices):
    raise NotImplementedError("write your pallas kernel here")


# speedup vs baseline: 4.2148x; 4.2148x over previous
import functools

import jax
import jax.numpy as jnp
from jax import lax
from jax.experimental import pallas as pl
from jax.experimental.pallas import tpu as pltpu
from jax.experimental.pallas import tpu_sc as plsc

_KS = 2
_STRIDE = 2
_L = 16


def _build_sc_scatter(BC, M, N, num_cores, num_subcores):
    NW = num_cores * num_subcores
    assert BC % NW == 0
    ROWS = BC // NW
    PASSES = 5
    QS = ((N + PASSES - 1) // PASSES + _L - 1) // _L * _L
    assert QS % 8 == 0
    mesh = plsc.VectorSubcoreMesh(
        core_axis_name="c",
        subcore_axis_name="s",
        num_cores=num_cores,
        num_subcores=num_subcores,
    )

    @functools.partial(
        pl.kernel,
        out_type=jax.ShapeDtypeStruct((BC * N,), jnp.float32),
        mesh=mesh,
        scratch_types=[
            pltpu.VMEM((M + _L,), jnp.int32),
            pltpu.VMEM((M,), jnp.float32),
            pltpu.VMEM((QS,), jnp.float32),
        ],
        compiler_params=pltpu.CompilerParams(needs_layout_passes=False),
    )
    def k(keys_hbm, vals_hbm, out_hbm, irow, vrow, qbuf):
        wid = lax.axis_index("s") * num_cores + lax.axis_index("c")

        def row_body(rr, carry):
            r = wid * ROWS + rr
            pltpu.sync_copy(keys_hbm.at[pl.ds(r * M, M)], irow.at[pl.ds(0, M)])
            pltpu.sync_copy(vals_hbm.at[pl.ds(r * M, M)], vrow)
            irow[pl.ds(M, _L)] = jnp.full((_L,), -1, jnp.int32)
            zeros = jnp.zeros((_L,), jnp.float32)
            for p in range(PASSES):
                lo = p * QS
                span = min(QS, N - lo)
                span = (span + _L - 1) // _L * _L

                def zero_body(j, c):
                    qbuf[pl.ds(pl.multiple_of(j * _L, _L), _L)] = zeros
                    return c

                lax.fori_loop(0, span // _L, zero_body, 0)

                def scat_body(i, c):
                    off = pl.multiple_of(i * _L, _L)
                    kv = irow[pl.ds(off, _L)]
                    kn = irow[pl.ds(off + 1, _L)]
                    vv = vrow[pl.ds(off, _L)]
                    lv = kv & (N - 1)
                    m = (lv >= lo) & (lv < lo + span) & (kv != kn)
                    loc = jnp.where(m, lv - lo, 0)
                    plsc.store_scatter(qbuf, [loc], vv, mask=m)
                    return c

                lax.fori_loop(0, M // _L, scat_body, 0)
                out_span = min(QS, N - lo)
                pltpu.sync_copy(
                    qbuf.at[pl.ds(0, out_span)],
                    out_hbm.at[pl.ds(r * N + lo, out_span)],
                )
            return carry

        lax.fori_loop(0, ROWS, row_body, 0)

    return k


def kernel(f_maps, indices):
    B, C, D, H, W = f_maps.shape
    Do = (D - 1) * _STRIDE + _KS
    Ho = (H - 1) * _STRIDE + _KS
    Wo = (W - 1) * _STRIDE + _KS
    N = Do * Ho * Wo
    M = D * H * W
    BC = B * C
    gkeys = (
        jnp.arange(BC, dtype=jnp.int32)[:, None] * N + indices.reshape(BC, M)
    ).reshape(-1)
    skeys, svals = lax.sort(
        (gkeys, f_maps.reshape(-1)), dimension=0, num_keys=1, is_stable=False
    )
    info = plsc.get_sparse_core_info()
    fn = _build_sc_scatter(BC, M, N, info.num_cores, info.num_subcores)
    out = fn(skeys, svals)
    return out.reshape(B, C, Do, Ho, Wo)


# speedup vs baseline: 4.9527x; 1.1751x over previous
import functools

import jax
import jax.numpy as jnp
from jax import lax
from jax.experimental import pallas as pl
from jax.experimental.pallas import tpu as pltpu
from jax.experimental.pallas import tpu_sc as plsc

_KS = 2
_STRIDE = 2
_L = 16


def _build_sc_scatter(BC, M, N, num_cores, num_subcores):
    NW = num_cores * num_subcores
    assert BC % NW == 0
    ROWS = BC // NW
    PASSES = 5
    _AL = _L * 8
    QS = ((N + PASSES - 1) // PASSES + _AL - 1) // _AL * _AL
    assert QS % 8 == 0 and (N - (PASSES - 1) * QS) % _AL == 0
    mesh = plsc.VectorSubcoreMesh(
        core_axis_name="c",
        subcore_axis_name="s",
        num_cores=num_cores,
        num_subcores=num_subcores,
    )

    @functools.partial(
        pl.kernel,
        out_type=jax.ShapeDtypeStruct((BC * N,), jnp.float32),
        mesh=mesh,
        scratch_types=[
            pltpu.VMEM((M + _L,), jnp.int32),
            pltpu.VMEM((M,), jnp.float32),
            pltpu.VMEM((QS,), jnp.float32),
        ],
        compiler_params=pltpu.CompilerParams(needs_layout_passes=False),
    )
    def k(keys_hbm, vals_hbm, out_hbm, irow, vrow, qbuf):
        wid = lax.axis_index("s") * num_cores + lax.axis_index("c")

        def row_body(rr, carry):
            r = wid * ROWS + rr
            pltpu.sync_copy(keys_hbm.at[pl.ds(r * M, M)], irow.at[pl.ds(0, M)])
            pltpu.sync_copy(vals_hbm.at[pl.ds(r * M, M)], vrow)
            irow[pl.ds(M, _L)] = jnp.full((_L,), -1, jnp.int32)
            zeros = jnp.zeros((_L,), jnp.float32)

            NV = M // _L

            def lower_bound(target):
                def bbody(_, lh):
                    blo, bhi = lh
                    mid = (blo + bhi) // 2
                    v = irow[pl.ds(pl.multiple_of(mid * _L, _L), _L)]
                    lv = v[0] & (N - 1)
                    right = lv < target
                    return (
                        jnp.where(right, mid + 1, blo),
                        jnp.where(right, bhi, mid),
                    )

                blo, _ = lax.fori_loop(
                    0, 11, bbody, (jnp.int32(0), jnp.int32(NV))
                )
                return blo

            bounds = [jnp.int32(0)]
            for p in range(1, PASSES):
                bounds.append(lower_bound(p * QS))
            bounds.append(jnp.int32(NV))

            for p in range(PASSES):
                lo = p * QS
                span = min(QS, N - lo)
                span = (span + _L - 1) // _L * _L

                @functools.partial(
                    plsc.parallel_loop, 0, span // _L, unroll=8
                )
                def _(j):
                    qbuf[pl.ds(pl.multiple_of(j * _L, _L), _L)] = zeros

                v0 = jnp.bitwise_and(jnp.maximum(bounds[p] - 1, 0), -4)
                v1 = jnp.minimum(
                    jnp.bitwise_and(bounds[p + 1] + 1 + 3, -4), NV
                )

                @functools.partial(plsc.parallel_loop, v0, v1, unroll=4)
                def _(i):
                    off = pl.multiple_of(i * _L, _L)
                    kv = irow[pl.ds(off, _L)]
                    kn = irow[pl.ds(off + 1, _L)]
                    vv = vrow[pl.ds(off, _L)]
                    lv = kv & (N - 1)
                    m = (lv >= lo) & (lv < lo + span) & (kv != kn)
                    loc = jnp.where(m, lv - lo, 0)
                    plsc.store_scatter(qbuf, [loc], vv, mask=m)

                out_span = min(QS, N - lo)
                pltpu.sync_copy(
                    qbuf.at[pl.ds(0, out_span)],
                    out_hbm.at[pl.ds(r * N + lo, out_span)],
                )
            return carry

        lax.fori_loop(0, ROWS, row_body, 0)

    return k


def kernel(f_maps, indices):
    B, C, D, H, W = f_maps.shape
    Do = (D - 1) * _STRIDE + _KS
    Ho = (H - 1) * _STRIDE + _KS
    Wo = (W - 1) * _STRIDE + _KS
    N = Do * Ho * Wo
    M = D * H * W
    BC = B * C
    gkeys = (
        jnp.arange(BC, dtype=jnp.int32)[:, None] * N + indices.reshape(BC, M)
    ).reshape(-1)
    skeys, svals = lax.sort(
        (gkeys, f_maps.reshape(-1)), dimension=0, num_keys=1, is_stable=False
    )
    info = plsc.get_sparse_core_info()
    fn = _build_sc_scatter(BC, M, N, info.num_cores, info.num_subcores)
    out = fn(skeys, svals)
    return out.reshape(B, C, Do, Ho, Wo)
